# baseline (speedup 1.0000x reference)
import functools

import jax
import jax.numpy as jnp
from jax import lax
from jax.experimental import pallas as pl
from jax.experimental.pallas import tpu as pltpu
from jax.experimental.pallas import tpu_sc as plsc

_N = 4096
_K = 8
_D = 128
_H = 64
_G = 4 * _H
_NW = 32
_BA = 512
_BB = 1024
_BC = 512
_CH = 512
_ROWS = _K * _N
_PW = _ROWS // _NW

_pc = pl.pallas_call


def _dot_t(x, w):
    return lax.dot_general(x, w, (((1,), (1,)), ((), ())),
                           preferred_element_type=jnp.float32)


def _scale_w(w):
    r = lax.broadcasted_iota(jnp.int32, w.shape, 0)
    keep = (r >= 2 * _H) & (r < 3 * _H)
    return jnp.where(keep, w, 0.5 * w)


def _scale_b(b):
    r = lax.broadcasted_iota(jnp.int32, b.shape, 1)
    keep = (r >= 2 * _H) & (r < 3 * _H)
    return jnp.where(keep, b, 0.5 * b)


def _cell2(g, c):
    t = jnp.tanh(g)
    u = 0.5 * t + 0.5
    i2 = jnp.concatenate([u[:, :_H], u[:, _G:_G + _H]], axis=1)
    f2 = jnp.concatenate([u[:, _H:2 * _H], u[:, _G + _H:_G + 2 * _H]], axis=1)
    g2 = jnp.concatenate([t[:, 2 * _H:3 * _H],
                          t[:, _G + 2 * _H:_G + 3 * _H]], axis=1)
    o2 = jnp.concatenate([u[:, 3 * _H:_G], u[:, _G + 3 * _H:]], axis=1)
    c = f2 * c + i2 * g2
    return o2 * jnp.tanh(c), c


def _cell20(g):
    t = jnp.tanh(g)
    u = 0.5 * t + 0.5
    i2 = jnp.concatenate([u[:, :_H], u[:, _G:_G + _H]], axis=1)
    g2 = jnp.concatenate([t[:, 2 * _H:3 * _H],
                          t[:, _G + 2 * _H:_G + 3 * _H]], axis=1)
    o2 = jnp.concatenate([u[:, 3 * _H:_G], u[:, _G + 3 * _H:]], axis=1)
    c = i2 * g2
    return o2 * jnp.tanh(c), c


def _content_body(x0a, x1a, x0b, x1b, x0c, x1c,
                  wf_r, whf_r, bf_r, wr_r, whr_r, br_r, out_r):
    wf = _scale_w(wf_r[...])
    whf = _scale_w(whf_r[...])
    bf = _scale_b(bf_r[...])
    wr = _scale_w(wr_r[...])
    whr = _scale_w(whr_r[...])
    br = _scale_b(br_r[...])
    for n, (x0r, x1r) in enumerate(((x0a, x1a), (x0b, x1b), (x0c, x1c))):
        x0 = x0r[...]
        x1 = x1r[...]
        g0 = jnp.concatenate([_dot_t(x0, wf) + bf, _dot_t(x1, wr) + br],
                             axis=1)
        h1, c1 = _cell20(g0)
        g1 = jnp.concatenate(
            [_dot_t(x1, wf) + bf + _dot_t(h1[:, :_H], whf),
             _dot_t(x0, wr) + br + _dot_t(h1[:, _H:], whr)], axis=1)
        h2, _ = _cell2(g1, c1)
        out_r[n] = 0.5 * (h1 + h2)


def _content_call(h_pairs, wf, whf, bf, wr, whr, br):
    full = lambda s: pl.BlockSpec(s, lambda i: tuple(0 for _ in s))
    return _pc(
        _content_body,
        grid=(_N // _BA,),
        in_specs=[pl.BlockSpec((_BA, _D), lambda i: (i, 0))] * 6 + [
            full((_G, _D)), full((_G, _H)), full((1, _G)),
            full((_G, _D)), full((_G, _H)), full((1, _G)),
        ],
        out_specs=pl.BlockSpec((3, _BA, _D), lambda i: (0, i, 0)),
        out_shape=jax.ShapeDtypeStruct((3, _N, _D), jnp.float32),
    )(*h_pairs, wf, whf, bf, wr, whr, br)


def _sc_gather(table, idx):
    mesh = plsc.VectorSubcoreMesh(core_axis_name="c", subcore_axis_name="s")

    @functools.partial(
        pl.kernel, mesh=mesh,
        out_type=jax.ShapeDtypeStruct((_ROWS, _D), jnp.float32),
        scratch_types=[
            pltpu.VMEM((_CH,), jnp.int32),
            pltpu.VMEM((_CH, _D), jnp.float32),
            pltpu.SemaphoreType.DMA,
        ],
    )
    def gk(table_hbm, idx_hbm, out_hbm, idx_v, rows_v, sem):
        wid = lax.axis_index("s") * 2 + lax.axis_index("c")
        base = wid * _PW

        def body(ci, carry):
            off = base + ci * _CH
            pltpu.sync_copy(idx_hbm.at[pl.ds(off, _CH)], idx_v)
            pltpu.async_copy(table_hbm.at[idx_v], rows_v, sem).wait()
            pltpu.sync_copy(rows_v, out_hbm.at[pl.ds(off, _CH)])
            return carry

        lax.fori_loop(0, _PW // _CH, body, 0)

    return gk(table, idx)


def _neigh_body(m_r, wf_r, whf_r, bf_r, wr_r, whr_r, br_r, out_r):
    m = m_r[...].reshape(_K * _BB, _D)
    wf = _scale_w(wf_r[...])
    whf = _scale_w(whf_r[...])
    bf = _scale_b(bf_r[...])
    wr = _scale_w(wr_r[...])
    whr = _scale_w(whr_r[...])
    br = _scale_b(br_r[...])
    gxf = _dot_t(m, wf) + bf
    gxr = _dot_t(m, wr) + br

    g0 = jnp.concatenate([gxf[:_BB], gxr[(_K - 1) * _BB:]], axis=1)
    h, c = _cell20(g0)
    acc = h
    for t in range(1, _K):
        g = jnp.concatenate(
            [gxf[t * _BB:(t + 1) * _BB] + _dot_t(h[:, :_H], whf),
             gxr[(_K - 1 - t) * _BB:(_K - t) * _BB] + _dot_t(h[:, _H:], whr)],
            axis=1)
        h, c = _cell2(g, c)
        acc += h

    out_r[...] = acc * (1.0 / _K)


def _neigh_call(mailbox, wf, whf, bf, wr, whr, br):
    full = lambda s: pl.BlockSpec(s, lambda b: tuple(0 for _ in s))
    return _pc(
        _neigh_body,
        grid=(_N // _BB,),
        in_specs=[
            pl.BlockSpec((_K, _BB, _D), lambda b: (0, b, 0)),
            full((_G, _D)), full((_G, _H)), full((1, _G)),
            full((_G, _D)), full((_G, _H)), full((1, _G)),
        ],
        out_specs=pl.BlockSpec((_BB, _D), lambda b: (b, 0)),
        out_shape=jax.ShapeDtypeStruct((_N, _D), jnp.float32),
    )(mailbox, wf, whf, bf, wr, whr, br)


def _att_body(*refs):
    n_refs = refs[:9]
    c_r, w_r, b_r, out_r = refs[9:]
    for n in range(3):
        dh = c_r[n]
        e0 = n_refs[n][...]
        e1 = n_refs[3 + n][...]
        e2 = n_refs[6 + n][...]
        w = w_r[n]
        b = b_r[n][:, :1]
        w1 = w[:, :_D]
        w2 = w[:, _D:]
        sd = jnp.sum(dh * w1, axis=1, keepdims=True)

        def score(e):
            s = sd + jnp.sum(e * w2, axis=1, keepdims=True) + b
            return jnp.where(s >= 0, s, 0.01 * s)

        s0, s1, s2, s3 = score(e0), score(e1), score(e2), score(dh)
        mx = jnp.maximum(jnp.maximum(s0, s1), jnp.maximum(s2, s3))
        p0 = jnp.exp(s0 - mx)
        p1 = jnp.exp(s1 - mx)
        p2 = jnp.exp(s2 - mx)
        p3 = jnp.exp(s3 - mx)
        z = p0 + p1 + p2 + p3
        out_r[n] = (e0 * p0 + e1 * p1 + e2 * p2 + dh * p3) / z


def _att_call(neighs, content, attw, attb):
    blk = pl.BlockSpec((_BC, _D), lambda b: (b, 0))
    return _pc(
        _att_body,
        grid=(_N // _BC,),
        in_specs=[blk] * 9 + [
            pl.BlockSpec((3, _BC, _D), lambda b: (0, b, 0)),
            pl.BlockSpec((3, 1, 2 * _D), lambda b: (0, 0, 0)),
            pl.BlockSpec((3, 1, _D), lambda b: (0, 0, 0)),
        ],
        out_specs=pl.BlockSpec((3, _BC, _D), lambda b: (0, b, 0)),
        out_shape=jax.ShapeDtypeStruct((3, _N, _D), jnp.float32),
    )(*neighs, content, attw, attb)


def kernel(h_a_0, h_a_1, h_b_0, h_b_1, h_c_0, h_c_1,
           nbr_a_a, nbr_a_b, nbr_a_c, nbr_b_a, nbr_b_b, nbr_b_c,
           nbr_c_a, nbr_c_b, nbr_c_c,
           c_Wihf, c_Whhf, c_bihf, c_bhhf, c_Wihr, c_Whhr, c_bihr, c_bhhr,
           n_a_Wihf, n_a_Whhf, n_a_bihf, n_a_bhhf,
           n_a_Wihr, n_a_Whhr, n_a_bihr, n_a_bhhr,
           n_b_Wihf, n_b_Whhf, n_b_bihf, n_b_bhhf,
           n_b_Wihr, n_b_Whhr, n_b_bihr, n_b_bhhr,
           n_c_Wihf, n_c_Whhf, n_c_bihf, n_c_bhhf,
           n_c_Wihr, n_c_Whhr, n_c_bihr, n_c_bhhr,
           att_a_W, att_a_b, att_b_W, att_b_b, att_c_W, att_c_b):
    content = _content_call(
        (h_a_0, h_a_1, h_b_0, h_b_1, h_c_0, h_c_1),
        c_Wihf, c_Whhf, (c_bihf + c_bhhf).reshape(1, _G),
        c_Wihr, c_Whhr, (c_bihr + c_bhhr).reshape(1, _G))
    table = content.reshape(3 * _N, _D)

    nbrs = (nbr_a_a, nbr_a_b, nbr_a_c, nbr_b_a, nbr_b_b, nbr_b_c,
            nbr_c_a, nbr_c_b, nbr_c_c)
    nweights = ((n_a_Wihf, n_a_Whhf, n_a_bihf, n_a_bhhf,
                 n_a_Wihr, n_a_Whhr, n_a_bihr, n_a_bhhr),
                (n_b_Wihf, n_b_Whhf, n_b_bihf, n_b_bhhf,
                 n_b_Wihr, n_b_Whhr, n_b_bihr, n_b_bhhr),
                (n_c_Wihf, n_c_Whhf, n_c_bihf, n_c_bhhf,
                 n_c_Wihr, n_c_Whhr, n_c_bihr, n_c_bhhr))

    neighs = []
    for e in range(9):
        s = e // 3
        idx = (nbrs[e].T + s * _N).reshape(_ROWS)
        mb = _sc_gather(table, idx).reshape(_K, _N, _D)
        wihf, whhf, bihf, bhhf, wihr, whhr, bihr, bhhr = nweights[s]
        neighs.append(_neigh_call(
            mb, wihf, whhf, (bihf + bhhf).reshape(1, _G),
            wihr, whhr, (bihr + bhhr).reshape(1, _G)))

    attw = jnp.stack([att_a_W, att_b_W, att_c_W])
    attb = jnp.broadcast_to(
        jnp.stack([att_a_b, att_b_b, att_c_b]).reshape(3, 1, 1), (3, 1, _D))
    return _att_call(neighs, content, attw, attb)


# speedup vs baseline: 1.0258x; 1.0258x over previous
import functools

import jax
import jax.numpy as jnp
from jax import lax
from jax.experimental import pallas as pl
from jax.experimental.pallas import tpu as pltpu
from jax.experimental.pallas import tpu_sc as plsc

_N = 4096
_K = 8
_D = 128
_H = 64
_G = 4 * _H
_NW = 32
_BA = 512
_BB = 512
_BC = 512
_CH = 512
_ROWS = _K * _N
_PW = _ROWS // _NW

_pc = pl.pallas_call


def _dot_t(x, w):
    return lax.dot_general(x, w, (((1,), (1,)), ((), ())),
                           preferred_element_type=jnp.float32)


def _scale_w(w):
    r = lax.broadcasted_iota(jnp.int32, w.shape, 0)
    keep = (r >= 2 * _H) & (r < 3 * _H)
    return jnp.where(keep, w, 0.5 * w)


def _scale_b(b):
    r = lax.broadcasted_iota(jnp.int32, b.shape, 1)
    keep = (r >= 2 * _H) & (r < 3 * _H)
    return jnp.where(keep, b, 0.5 * b)


def _cell2(g, c):
    t = jnp.tanh(g)
    u = 0.5 * t + 0.5
    i2 = jnp.concatenate([u[:, :_H], u[:, _G:_G + _H]], axis=1)
    f2 = jnp.concatenate([u[:, _H:2 * _H], u[:, _G + _H:_G + 2 * _H]], axis=1)
    g2 = jnp.concatenate([t[:, 2 * _H:3 * _H],
                          t[:, _G + 2 * _H:_G + 3 * _H]], axis=1)
    o2 = jnp.concatenate([u[:, 3 * _H:_G], u[:, _G + 3 * _H:]], axis=1)
    c = f2 * c + i2 * g2
    return o2 * jnp.tanh(c), c


def _cell20(g):
    t = jnp.tanh(g)
    u = 0.5 * t + 0.5
    i2 = jnp.concatenate([u[:, :_H], u[:, _G:_G + _H]], axis=1)
    g2 = jnp.concatenate([t[:, 2 * _H:3 * _H],
                          t[:, _G + 2 * _H:_G + 3 * _H]], axis=1)
    o2 = jnp.concatenate([u[:, 3 * _H:_G], u[:, _G + 3 * _H:]], axis=1)
    c = i2 * g2
    return o2 * jnp.tanh(c), c


def _content_body(x0a, x1a, x0b, x1b, x0c, x1c,
                  wf_r, whf_r, bf_r, wr_r, whr_r, br_r, out_r):
    wf = _scale_w(wf_r[...])
    whf = _scale_w(whf_r[...])
    bf = _scale_b(bf_r[...])
    wr = _scale_w(wr_r[...])
    whr = _scale_w(whr_r[...])
    br = _scale_b(br_r[...])
    for n, (x0r, x1r) in enumerate(((x0a, x1a), (x0b, x1b), (x0c, x1c))):
        x0 = x0r[...]
        x1 = x1r[...]
        g0 = jnp.concatenate([_dot_t(x0, wf) + bf, _dot_t(x1, wr) + br],
                             axis=1)
        h1, c1 = _cell20(g0)
        g1 = jnp.concatenate(
            [_dot_t(x1, wf) + bf + _dot_t(h1[:, :_H], whf),
             _dot_t(x0, wr) + br + _dot_t(h1[:, _H:], whr)], axis=1)
        h2, _ = _cell2(g1, c1)
        out_r[n] = 0.5 * (h1 + h2)


def _content_call(h_pairs, wf, whf, bf, wr, whr, br):
    full = lambda s: pl.BlockSpec(s, lambda i: tuple(0 for _ in s))
    return _pc(
        _content_body,
        grid=(_N // _BA,),
        in_specs=[pl.BlockSpec((_BA, _D), lambda i: (i, 0))] * 6 + [
            full((_G, _D)), full((_G, _H)), full((1, _G)),
            full((_G, _D)), full((_G, _H)), full((1, _G)),
        ],
        out_specs=pl.BlockSpec((3, _BA, _D), lambda i: (0, i, 0)),
        out_shape=jax.ShapeDtypeStruct((3, _N, _D), jnp.float32),
    )(*h_pairs, wf, whf, bf, wr, whr, br)


def _sc_gather(table, idx):
    mesh = plsc.VectorSubcoreMesh(core_axis_name="c", subcore_axis_name="s")

    @functools.partial(
        pl.kernel, mesh=mesh,
        out_type=jax.ShapeDtypeStruct((_ROWS, _D), jnp.float32),
        scratch_types=[
            pltpu.VMEM((_CH,), jnp.int32),
            pltpu.VMEM((_CH, _D), jnp.float32),
            pltpu.SemaphoreType.DMA,
        ],
    )
    def gk(table_hbm, idx_hbm, out_hbm, idx_v, rows_v, sem):
        wid = lax.axis_index("s") * 2 + lax.axis_index("c")
        base = wid * _PW

        def body(ci, carry):
            off = base + ci * _CH
            pltpu.sync_copy(idx_hbm.at[pl.ds(off, _CH)], idx_v)
            pltpu.async_copy(table_hbm.at[idx_v], rows_v, sem).wait()
            pltpu.sync_copy(rows_v, out_hbm.at[pl.ds(off, _CH)])
            return carry

        lax.fori_loop(0, _PW // _CH, body, 0)

    return gk(table, idx)


def _neigh_body(m_r, wf_r, whf_r, bf_r, wr_r, whr_r, br_r, out_r):
    m = m_r[...].reshape(_K * _BB, _D)
    wf = _scale_w(wf_r[...])
    whf = _scale_w(whf_r[...])
    bf = _scale_b(bf_r[...])
    wr = _scale_w(wr_r[...])
    whr = _scale_w(whr_r[...])
    br = _scale_b(br_r[...])
    gxf = _dot_t(m, wf) + bf
    gxr = _dot_t(m, wr) + br

    g0 = jnp.concatenate([gxf[:_BB], gxr[(_K - 1) * _BB:]], axis=1)
    h, c = _cell20(g0)
    acc = h
    for t in range(1, _K):
        g = jnp.concatenate(
            [gxf[t * _BB:(t + 1) * _BB] + _dot_t(h[:, :_H], whf),
             gxr[(_K - 1 - t) * _BB:(_K - t) * _BB] + _dot_t(h[:, _H:], whr)],
            axis=1)
        h, c = _cell2(g, c)
        acc += h

    out_r[...] = acc * (1.0 / _K)


def _neigh_call(mailbox, wf, whf, bf, wr, whr, br):
    full = lambda s: pl.BlockSpec(s, lambda b: tuple(0 for _ in s))
    return _pc(
        _neigh_body,
        grid=(_N // _BB,),
        in_specs=[
            pl.BlockSpec((_K, _BB, _D), lambda b: (0, b, 0)),
            full((_G, _D)), full((_G, _H)), full((1, _G)),
            full((_G, _D)), full((_G, _H)), full((1, _G)),
        ],
        out_specs=pl.BlockSpec((_BB, _D), lambda b: (b, 0)),
        out_shape=jax.ShapeDtypeStruct((_N, _D), jnp.float32),
    )(mailbox, wf, whf, bf, wr, whr, br)


def _att_body(*refs):
    n_refs = refs[:9]
    c_r, w_r, b_r, out_r = refs[9:]
    for n in range(3):
        dh = c_r[n]
        e0 = n_refs[n][...]
        e1 = n_refs[3 + n][...]
        e2 = n_refs[6 + n][...]
        w = w_r[n]
        b = b_r[n][:, :1]
        w1 = w[:, :_D]
        w2 = w[:, _D:]
        sd = jnp.sum(dh * w1, axis=1, keepdims=True)

        def score(e):
            s = sd + jnp.sum(e * w2, axis=1, keepdims=True) + b
            return jnp.where(s >= 0, s, 0.01 * s)

        s0, s1, s2, s3 = score(e0), score(e1), score(e2), score(dh)
        mx = jnp.maximum(jnp.maximum(s0, s1), jnp.maximum(s2, s3))
        p0 = jnp.exp(s0 - mx)
        p1 = jnp.exp(s1 - mx)
        p2 = jnp.exp(s2 - mx)
        p3 = jnp.exp(s3 - mx)
        z = p0 + p1 + p2 + p3
        out_r[n] = (e0 * p0 + e1 * p1 + e2 * p2 + dh * p3) / z


def _att_call(neighs, content, attw, attb):
    blk = pl.BlockSpec((_BC, _D), lambda b: (b, 0))
    return _pc(
        _att_body,
        grid=(_N // _BC,),
        in_specs=[blk] * 9 + [
            pl.BlockSpec((3, _BC, _D), lambda b: (0, b, 0)),
            pl.BlockSpec((3, 1, 2 * _D), lambda b: (0, 0, 0)),
            pl.BlockSpec((3, 1, _D), lambda b: (0, 0, 0)),
        ],
        out_specs=pl.BlockSpec((3, _BC, _D), lambda b: (0, b, 0)),
        out_shape=jax.ShapeDtypeStruct((3, _N, _D), jnp.float32),
    )(*neighs, content, attw, attb)


def kernel(h_a_0, h_a_1, h_b_0, h_b_1, h_c_0, h_c_1,
           nbr_a_a, nbr_a_b, nbr_a_c, nbr_b_a, nbr_b_b, nbr_b_c,
           nbr_c_a, nbr_c_b, nbr_c_c,
           c_Wihf, c_Whhf, c_bihf, c_bhhf, c_Wihr, c_Whhr, c_bihr, c_bhhr,
           n_a_Wihf, n_a_Whhf, n_a_bihf, n_a_bhhf,
           n_a_Wihr, n_a_Whhr, n_a_bihr, n_a_bhhr,
           n_b_Wihf, n_b_Whhf, n_b_bihf, n_b_bhhf,
           n_b_Wihr, n_b_Whhr, n_b_bihr, n_b_bhhr,
           n_c_Wihf, n_c_Whhf, n_c_bihf, n_c_bhhf,
           n_c_Wihr, n_c_Whhr, n_c_bihr, n_c_bhhr,
           att_a_W, att_a_b, att_b_W, att_b_b, att_c_W, att_c_b):
    content = _content_call(
        (h_a_0, h_a_1, h_b_0, h_b_1, h_c_0, h_c_1),
        c_Wihf, c_Whhf, (c_bihf + c_bhhf).reshape(1, _G),
        c_Wihr, c_Whhr, (c_bihr + c_bhhr).reshape(1, _G))
    table = content.reshape(3 * _N, _D)

    nbrs = (nbr_a_a, nbr_a_b, nbr_a_c, nbr_b_a, nbr_b_b, nbr_b_c,
            nbr_c_a, nbr_c_b, nbr_c_c)
    nweights = ((n_a_Wihf, n_a_Whhf, n_a_bihf, n_a_bhhf,
                 n_a_Wihr, n_a_Whhr, n_a_bihr, n_a_bhhr),
                (n_b_Wihf, n_b_Whhf, n_b_bihf, n_b_bhhf,
                 n_b_Wihr, n_b_Whhr, n_b_bihr, n_b_bhhr),
                (n_c_Wihf, n_c_Whhf, n_c_bihf, n_c_bhhf,
                 n_c_Wihr, n_c_Whhr, n_c_bihr, n_c_bhhr))

    neighs = []
    for e in range(9):
        s = e // 3
        idx = (nbrs[e].T + s * _N).reshape(_ROWS)
        mb = _sc_gather(table, idx).reshape(_K, _N, _D)
        wihf, whhf, bihf, bhhf, wihr, whhr, bihr, bhhr = nweights[s]
        neighs.append(_neigh_call(
            mb, wihf, whhf, (bihf + bhhf).reshape(1, _G),
            wihr, whhr, (bihr + bhhr).reshape(1, _G)))

    attw = jnp.stack([att_a_W, att_b_W, att_c_W])
    attb = jnp.broadcast_to(
        jnp.stack([att_a_b, att_b_b, att_c_b]).reshape(3, 1, 1), (3, 1, _D))
    return _att_call(neighs, content, attw, attb)


# speedup vs baseline: 1.0286x; 1.0027x over previous
import functools

import jax
import jax.numpy as jnp
from jax import lax
from jax.experimental import pallas as pl
from jax.experimental.pallas import tpu as pltpu
from jax.experimental.pallas import tpu_sc as plsc

_N = 4096
_K = 8
_D = 128
_H = 64
_G = 4 * _H
_NW = 32
_BA = 512
_BB = 512
_BC = 512
_CH = 512
_ROWS = _K * _N
_PW = _ROWS // _NW

_pc = pl.pallas_call


def _dot_t(x, w):
    return lax.dot_general(x, w, (((1,), (1,)), ((), ())),
                           preferred_element_type=jnp.float32)


def _scale_w(w):
    r = lax.broadcasted_iota(jnp.int32, w.shape, 0)
    keep = (r >= 2 * _H) & (r < 3 * _H)
    return jnp.where(keep, w, 0.5 * w)


def _scale_b(b):
    r = lax.broadcasted_iota(jnp.int32, b.shape, 1)
    keep = (r >= 2 * _H) & (r < 3 * _H)
    return jnp.where(keep, b, 0.5 * b)


def _cell2(g, c):
    t = jnp.tanh(g)
    u = 0.5 * t + 0.5
    i2 = jnp.concatenate([u[:, :_H], u[:, _G:_G + _H]], axis=1)
    f2 = jnp.concatenate([u[:, _H:2 * _H], u[:, _G + _H:_G + 2 * _H]], axis=1)
    g2 = jnp.concatenate([t[:, 2 * _H:3 * _H],
                          t[:, _G + 2 * _H:_G + 3 * _H]], axis=1)
    o2 = jnp.concatenate([u[:, 3 * _H:_G], u[:, _G + 3 * _H:]], axis=1)
    c = f2 * c + i2 * g2
    return o2 * jnp.tanh(c), c


def _cell20(g):
    t = jnp.tanh(g)
    u = 0.5 * t + 0.5
    i2 = jnp.concatenate([u[:, :_H], u[:, _G:_G + _H]], axis=1)
    g2 = jnp.concatenate([t[:, 2 * _H:3 * _H],
                          t[:, _G + 2 * _H:_G + 3 * _H]], axis=1)
    o2 = jnp.concatenate([u[:, 3 * _H:_G], u[:, _G + 3 * _H:]], axis=1)
    c = i2 * g2
    return o2 * jnp.tanh(c), c


def _content_body(x0a, x1a, x0b, x1b, x0c, x1c,
                  wf_r, whf_r, bf_r, wr_r, whr_r, br_r, out_r):
    wf = _scale_w(wf_r[...])
    whf = _scale_w(whf_r[...])
    bf = _scale_b(bf_r[...])
    wr = _scale_w(wr_r[...])
    whr = _scale_w(whr_r[...])
    br = _scale_b(br_r[...])
    for n, (x0r, x1r) in enumerate(((x0a, x1a), (x0b, x1b), (x0c, x1c))):
        x0 = x0r[...]
        x1 = x1r[...]
        g0 = jnp.concatenate([_dot_t(x0, wf) + bf, _dot_t(x1, wr) + br],
                             axis=1)
        h1, c1 = _cell20(g0)
        g1 = jnp.concatenate(
            [_dot_t(x1, wf) + bf + _dot_t(h1[:, :_H], whf),
             _dot_t(x0, wr) + br + _dot_t(h1[:, _H:], whr)], axis=1)
        h2, _ = _cell2(g1, c1)
        out_r[n] = 0.5 * (h1 + h2)


def _content_call(h_pairs, wf, whf, bf, wr, whr, br):
    full = lambda s: pl.BlockSpec(s, lambda i: tuple(0 for _ in s))
    return _pc(
        _content_body,
        grid=(_N // _BA,),
        in_specs=[pl.BlockSpec((_BA, _D), lambda i: (i, 0))] * 6 + [
            full((_G, _D)), full((_G, _H)), full((1, _G)),
            full((_G, _D)), full((_G, _H)), full((1, _G)),
        ],
        out_specs=pl.BlockSpec((3, _BA, _D), lambda i: (0, i, 0)),
        out_shape=jax.ShapeDtypeStruct((3, _N, _D), jnp.float32),
    )(*h_pairs, wf, whf, bf, wr, whr, br)


def _sc_gather(table, idx):
    mesh = plsc.VectorSubcoreMesh(core_axis_name="c", subcore_axis_name="s")

    @functools.partial(
        pl.kernel, mesh=mesh,
        out_type=jax.ShapeDtypeStruct((_ROWS, _D), jnp.float32),
        scratch_types=[
            pltpu.VMEM((_CH,), jnp.int32),
            pltpu.VMEM((_CH, _D), jnp.float32),
            pltpu.SemaphoreType.DMA,
        ],
    )
    def gk(table_hbm, idx_hbm, out_hbm, idx_v, rows_v, sem):
        wid = lax.axis_index("s") * 2 + lax.axis_index("c")
        base = wid * _PW

        def body(ci, carry):
            off = base + ci * _CH
            pltpu.sync_copy(idx_hbm.at[pl.ds(off, _CH)], idx_v)
            pltpu.async_copy(table_hbm.at[idx_v], rows_v, sem).wait()
            pltpu.sync_copy(rows_v, out_hbm.at[pl.ds(off, _CH)])
            return carry

        lax.fori_loop(0, _PW // _CH, body, 0)

    return gk(table, idx)


def _neigh_body(m_r, wf_r, whf_r, bf_r, wr_r, whr_r, br_r, out_r):
    m = m_r[...].reshape(_K * _BB, _D)
    wf = _scale_w(wf_r[...])
    whf = _scale_w(whf_r[...])
    bf = _scale_b(bf_r[...])
    wr = _scale_w(wr_r[...])
    whr = _scale_w(whr_r[...])
    br = _scale_b(br_r[...])
    gxf = _dot_t(m, wf) + bf
    gxr = _dot_t(m, wr) + br

    g0 = jnp.concatenate([gxf[:_BB], gxr[(_K - 1) * _BB:]], axis=1)
    h, c = _cell20(g0)
    acc = h
    for t in range(1, _K):
        g = jnp.concatenate(
            [gxf[t * _BB:(t + 1) * _BB] + _dot_t(h[:, :_H], whf),
             gxr[(_K - 1 - t) * _BB:(_K - t) * _BB] + _dot_t(h[:, _H:], whr)],
            axis=1)
        h, c = _cell2(g, c)
        acc += h

    out_r[...] = acc * (1.0 / _K)


def _neigh_call(mailbox, wf, whf, bf, wr, whr, br):
    full = lambda s: pl.BlockSpec(s, lambda b: tuple(0 for _ in s))
    return _pc(
        _neigh_body,
        grid=(_N // _BB,),
        in_specs=[
            pl.BlockSpec((1, _K, _BB, _D), lambda b: (b, 0, 0, 0)),
            full((_G, _D)), full((_G, _H)), full((1, _G)),
            full((_G, _D)), full((_G, _H)), full((1, _G)),
        ],
        out_specs=pl.BlockSpec((_BB, _D), lambda b: (b, 0)),
        out_shape=jax.ShapeDtypeStruct((_N, _D), jnp.float32),
    )(mailbox, wf, whf, bf, wr, whr, br)


def _att_body(*refs):
    n_refs = refs[:9]
    c_r, w_r, b_r, out_r = refs[9:]
    for n in range(3):
        dh = c_r[n]
        e0 = n_refs[n][...]
        e1 = n_refs[3 + n][...]
        e2 = n_refs[6 + n][...]
        w = w_r[n]
        b = b_r[n][:, :1]
        w1 = w[:, :_D]
        w2 = w[:, _D:]
        sd = jnp.sum(dh * w1, axis=1, keepdims=True)

        def score(e):
            s = sd + jnp.sum(e * w2, axis=1, keepdims=True) + b
            return jnp.where(s >= 0, s, 0.01 * s)

        s0, s1, s2, s3 = score(e0), score(e1), score(e2), score(dh)
        mx = jnp.maximum(jnp.maximum(s0, s1), jnp.maximum(s2, s3))
        p0 = jnp.exp(s0 - mx)
        p1 = jnp.exp(s1 - mx)
        p2 = jnp.exp(s2 - mx)
        p3 = jnp.exp(s3 - mx)
        z = p0 + p1 + p2 + p3
        out_r[n] = (e0 * p0 + e1 * p1 + e2 * p2 + dh * p3) / z


def _att_call(neighs, content, attw, attb):
    blk = pl.BlockSpec((_BC, _D), lambda b: (b, 0))
    return _pc(
        _att_body,
        grid=(_N // _BC,),
        in_specs=[blk] * 9 + [
            pl.BlockSpec((3, _BC, _D), lambda b: (0, b, 0)),
            pl.BlockSpec((3, 1, 2 * _D), lambda b: (0, 0, 0)),
            pl.BlockSpec((3, 1, _D), lambda b: (0, 0, 0)),
        ],
        out_specs=pl.BlockSpec((3, _BC, _D), lambda b: (0, b, 0)),
        out_shape=jax.ShapeDtypeStruct((3, _N, _D), jnp.float32),
    )(*neighs, content, attw, attb)


def kernel(h_a_0, h_a_1, h_b_0, h_b_1, h_c_0, h_c_1,
           nbr_a_a, nbr_a_b, nbr_a_c, nbr_b_a, nbr_b_b, nbr_b_c,
           nbr_c_a, nbr_c_b, nbr_c_c,
           c_Wihf, c_Whhf, c_bihf, c_bhhf, c_Wihr, c_Whhr, c_bihr, c_bhhr,
           n_a_Wihf, n_a_Whhf, n_a_bihf, n_a_bhhf,
           n_a_Wihr, n_a_Whhr, n_a_bihr, n_a_bhhr,
           n_b_Wihf, n_b_Whhf, n_b_bihf, n_b_bhhf,
           n_b_Wihr, n_b_Whhr, n_b_bihr, n_b_bhhr,
           n_c_Wihf, n_c_Whhf, n_c_bihf, n_c_bhhf,
           n_c_Wihr, n_c_Whhr, n_c_bihr, n_c_bhhr,
           att_a_W, att_a_b, att_b_W, att_b_b, att_c_W, att_c_b):
    content = _content_call(
        (h_a_0, h_a_1, h_b_0, h_b_1, h_c_0, h_c_1),
        c_Wihf, c_Whhf, (c_bihf + c_bhhf).reshape(1, _G),
        c_Wihr, c_Whhr, (c_bihr + c_bhhr).reshape(1, _G))
    table = content.reshape(3 * _N, _D)

    nbrs = (nbr_a_a, nbr_a_b, nbr_a_c, nbr_b_a, nbr_b_b, nbr_b_c,
            nbr_c_a, nbr_c_b, nbr_c_c)
    nweights = ((n_a_Wihf, n_a_Whhf, n_a_bihf, n_a_bhhf,
                 n_a_Wihr, n_a_Whhr, n_a_bihr, n_a_bhhr),
                (n_b_Wihf, n_b_Whhf, n_b_bihf, n_b_bhhf,
                 n_b_Wihr, n_b_Whhr, n_b_bihr, n_b_bhhr),
                (n_c_Wihf, n_c_Whhf, n_c_bihf, n_c_bhhf,
                 n_c_Wihr, n_c_Whhr, n_c_bihr, n_c_bhhr))

    neighs = []
    for e in range(9):
        s = e // 3
        idx = (nbrs[e].reshape(_N // _BB, _BB, _K).transpose(0, 2, 1)
               + s * _N).reshape(_ROWS)
        mb = _sc_gather(table, idx).reshape(_N // _BB, _K, _BB, _D)
        wihf, whhf, bihf, bhhf, wihr, whhr, bihr, bhhr = nweights[s]
        neighs.append(_neigh_call(
            mb, wihf, whhf, (bihf + bhhf).reshape(1, _G),
            wihr, whhr, (bihr + bhhr).reshape(1, _G)))

    attw = jnp.stack([att_a_W, att_b_W, att_c_W])
    attb = jnp.broadcast_to(
        jnp.stack([att_a_b, att_b_b, att_c_b]).reshape(3, 1, 1), (3, 1, _D))
    return _att_call(neighs, content, attw, attb)


# speedup vs baseline: 1.0616x; 1.0321x over previous
import functools

import jax
import jax.numpy as jnp
from jax import lax
from jax.experimental import pallas as pl
from jax.experimental.pallas import tpu as pltpu
from jax.experimental.pallas import tpu_sc as plsc

_N = 4096
_K = 8
_D = 128
_H = 64
_G = 4 * _H
_NW = 32
_BA = 512
_BB = 512
_BC = 512
_CH = 512
_ROWS = _K * _N
_PW = _ROWS // _NW

_pc = pl.pallas_call


def _dot_t(x, w):
    return lax.dot_general(x, w, (((1,), (1,)), ((), ())),
                           preferred_element_type=jnp.float32)


def _scale_w(w):
    r = lax.broadcasted_iota(jnp.int32, w.shape, 0)
    keep = (r >= 2 * _H) & (r < 3 * _H)
    return jnp.where(keep, w, 0.5 * w)


def _scale_b(b):
    r = lax.broadcasted_iota(jnp.int32, b.shape, 1)
    keep = (r >= 2 * _H) & (r < 3 * _H)
    return jnp.where(keep, b, 0.5 * b)


def _cell(gs, c_prev):
    t = jnp.tanh(gs)
    u = 0.5 * t + 0.5
    c = u[:, _H:2 * _H] * c_prev + u[:, :_H] * t[:, 2 * _H:3 * _H]
    return u[:, 3 * _H:] * jnp.tanh(c), c


def _cell0(gs):
    t = jnp.tanh(gs)
    u = 0.5 * t + 0.5
    c = u[:, :_H] * t[:, 2 * _H:3 * _H]
    return u[:, 3 * _H:] * jnp.tanh(c), c


def _content_body(x0a, x1a, x0b, x1b, x0c, x1c,
                  wf_r, whf_r, bf_r, wr_r, whr_r, br_r, out_r):
    wf = _scale_w(wf_r[...])
    whf = _scale_w(whf_r[...])
    bf = _scale_b(bf_r[...])
    wr = _scale_w(wr_r[...])
    whr = _scale_w(whr_r[...])
    br = _scale_b(br_r[...])
    for n, (x0r, x1r) in enumerate(((x0a, x1a), (x0b, x1b), (x0c, x1c))):
        x0 = x0r[...]
        x1 = x1r[...]
        h1, c1 = _cell0(_dot_t(x0, wf) + bf)
        h2, _ = _cell(_dot_t(x1, wf) + _dot_t(h1, whf) + bf, c1)
        b1, cb1 = _cell0(_dot_t(x1, wr) + br)
        b2, _ = _cell(_dot_t(x0, wr) + _dot_t(b1, whr) + br, cb1)
        out_r[n] = jnp.concatenate([0.5 * (h1 + h2), 0.5 * (b1 + b2)],
                                   axis=1)


def _content_call(h_pairs, wf, whf, bf, wr, whr, br):
    full = lambda s: pl.BlockSpec(s, lambda i: tuple(0 for _ in s))
    return _pc(
        _content_body,
        grid=(_N // _BA,),
        in_specs=[pl.BlockSpec((_BA, _D), lambda i: (i, 0))] * 6 + [
            full((_G, _D)), full((_G, _H)), full((1, _G)),
            full((_G, _D)), full((_G, _H)), full((1, _G)),
        ],
        out_specs=pl.BlockSpec((3, _BA, _D), lambda i: (0, i, 0)),
        out_shape=jax.ShapeDtypeStruct((3, _N, _D), jnp.float32),
    )(*h_pairs, wf, whf, bf, wr, whr, br)


def _sc_gather(table, idx):
    mesh = plsc.VectorSubcoreMesh(core_axis_name="c", subcore_axis_name="s")

    @functools.partial(
        pl.kernel, mesh=mesh,
        out_type=jax.ShapeDtypeStruct((_ROWS, _D), jnp.float32),
        scratch_types=[
            pltpu.VMEM((_CH,), jnp.int32),
            pltpu.VMEM((_CH, _D), jnp.float32),
            pltpu.SemaphoreType.DMA,
        ],
    )
    def gk(table_hbm, idx_hbm, out_hbm, idx_v, rows_v, sem):
        wid = lax.axis_index("s") * 2 + lax.axis_index("c")
        base = wid * _PW

        def body(ci, carry):
            off = base + ci * _CH
            pltpu.sync_copy(idx_hbm.at[pl.ds(off, _CH)], idx_v)
            pltpu.async_copy(table_hbm.at[idx_v], rows_v, sem).wait()
            pltpu.sync_copy(rows_v, out_hbm.at[pl.ds(off, _CH)])
            return carry

        lax.fori_loop(0, _PW // _CH, body, 0)

    return gk(table, idx)


def _neigh_body(m_r, wf_r, whf_r, bf_r, wr_r, whr_r, br_r, out_r):
    m = m_r[...].reshape(_K * _BB, _D)
    wf = _scale_w(wf_r[...])
    whf = _scale_w(whf_r[...])
    bf = _scale_b(bf_r[...])
    wr = _scale_w(wr_r[...])
    whr = _scale_w(whr_r[...])
    br = _scale_b(br_r[...])
    gxf = _dot_t(m, wf) + bf
    gxr = _dot_t(m, wr) + br

    h, c = _cell0(gxf[:_BB])
    accf = h
    for t in range(1, _K):
        h, c = _cell(gxf[t * _BB:(t + 1) * _BB] + _dot_t(h, whf), c)
        accf += h

    h, c = _cell0(gxr[(_K - 1) * _BB:])
    accr = h
    for t in range(_K - 2, -1, -1):
        h, c = _cell(gxr[t * _BB:(t + 1) * _BB] + _dot_t(h, whr), c)
        accr += h

    out_r[...] = jnp.concatenate([accf, accr], axis=1) * (1.0 / _K)


def _neigh_call(mailbox, wf, whf, bf, wr, whr, br):
    full = lambda s: pl.BlockSpec(s, lambda b: tuple(0 for _ in s))
    return _pc(
        _neigh_body,
        grid=(_N // _BB,),
        in_specs=[
            pl.BlockSpec((1, _K, _BB, _D), lambda b: (b, 0, 0, 0)),
            full((_G, _D)), full((_G, _H)), full((1, _G)),
            full((_G, _D)), full((_G, _H)), full((1, _G)),
        ],
        out_specs=pl.BlockSpec((_BB, _D), lambda b: (b, 0)),
        out_shape=jax.ShapeDtypeStruct((_N, _D), jnp.float32),
    )(mailbox, wf, whf, bf, wr, whr, br)


def _att_body(*refs):
    n_refs = refs[:9]
    c_r, w_r, b_r, out_r = refs[9:]
    for n in range(3):
        dh = c_r[n]
        e0 = n_refs[n][...]
        e1 = n_refs[3 + n][...]
        e2 = n_refs[6 + n][...]
        w = w_r[n]
        b = b_r[n][:, :1]
        w1 = w[:, :_D]
        w2 = w[:, _D:]
        sd = jnp.sum(dh * w1, axis=1, keepdims=True)

        def score(e):
            s = sd + jnp.sum(e * w2, axis=1, keepdims=True) + b
            return jnp.where(s >= 0, s, 0.01 * s)

        s0, s1, s2, s3 = score(e0), score(e1), score(e2), score(dh)
        mx = jnp.maximum(jnp.maximum(s0, s1), jnp.maximum(s2, s3))
        p0 = jnp.exp(s0 - mx)
        p1 = jnp.exp(s1 - mx)
        p2 = jnp.exp(s2 - mx)
        p3 = jnp.exp(s3 - mx)
        z = p0 + p1 + p2 + p3
        out_r[n] = (e0 * p0 + e1 * p1 + e2 * p2 + dh * p3) / z


def _att_call(neighs, content, attw, attb):
    blk = pl.BlockSpec((_BC, _D), lambda b: (b, 0))
    return _pc(
        _att_body,
        grid=(_N // _BC,),
        in_specs=[blk] * 9 + [
            pl.BlockSpec((3, _BC, _D), lambda b: (0, b, 0)),
            pl.BlockSpec((3, 1, 2 * _D), lambda b: (0, 0, 0)),
            pl.BlockSpec((3, 1, _D), lambda b: (0, 0, 0)),
        ],
        out_specs=pl.BlockSpec((3, _BC, _D), lambda b: (0, b, 0)),
        out_shape=jax.ShapeDtypeStruct((3, _N, _D), jnp.float32),
    )(*neighs, content, attw, attb)


def kernel(h_a_0, h_a_1, h_b_0, h_b_1, h_c_0, h_c_1,
           nbr_a_a, nbr_a_b, nbr_a_c, nbr_b_a, nbr_b_b, nbr_b_c,
           nbr_c_a, nbr_c_b, nbr_c_c,
           c_Wihf, c_Whhf, c_bihf, c_bhhf, c_Wihr, c_Whhr, c_bihr, c_bhhr,
           n_a_Wihf, n_a_Whhf, n_a_bihf, n_a_bhhf,
           n_a_Wihr, n_a_Whhr, n_a_bihr, n_a_bhhr,
           n_b_Wihf, n_b_Whhf, n_b_bihf, n_b_bhhf,
           n_b_Wihr, n_b_Whhr, n_b_bihr, n_b_bhhr,
           n_c_Wihf, n_c_Whhf, n_c_bihf, n_c_bhhf,
           n_c_Wihr, n_c_Whhr, n_c_bihr, n_c_bhhr,
           att_a_W, att_a_b, att_b_W, att_b_b, att_c_W, att_c_b):
    content = _content_call(
        (h_a_0, h_a_1, h_b_0, h_b_1, h_c_0, h_c_1),
        c_Wihf, c_Whhf, (c_bihf + c_bhhf).reshape(1, _G),
        c_Wihr, c_Whhr, (c_bihr + c_bhhr).reshape(1, _G))
    table = content.reshape(3 * _N, _D)

    nbrs = (nbr_a_a, nbr_a_b, nbr_a_c, nbr_b_a, nbr_b_b, nbr_b_c,
            nbr_c_a, nbr_c_b, nbr_c_c)
    nweights = ((n_a_Wihf, n_a_Whhf, n_a_bihf, n_a_bhhf,
                 n_a_Wihr, n_a_Whhr, n_a_bihr, n_a_bhhr),
                (n_b_Wihf, n_b_Whhf, n_b_bihf, n_b_bhhf,
                 n_b_Wihr, n_b_Whhr, n_b_bihr, n_b_bhhr),
                (n_c_Wihf, n_c_Whhf, n_c_bihf, n_c_bhhf,
                 n_c_Wihr, n_c_Whhr, n_c_bihr, n_c_bhhr))

    neighs = []
    for e in range(9):
        s = e // 3
        idx = (nbrs[e].reshape(_N // _BB, _BB, _K).transpose(0, 2, 1)
               + s * _N).reshape(_ROWS)
        mb = _sc_gather(table, idx).reshape(_N // _BB, _K, _BB, _D)
        wihf, whhf, bihf, bhhf, wihr, whhr, bihr, bhhr = nweights[s]
        neighs.append(_neigh_call(
            mb, wihf, whhf, (bihf + bhhf).reshape(1, _G),
            wihr, whhr, (bihr + bhhr).reshape(1, _G)))

    attw = jnp.stack([att_a_W, att_b_W, att_c_W])
    attb = jnp.broadcast_to(
        jnp.stack([att_a_b, att_b_b, att_c_b]).reshape(3, 1, 1), (3, 1, _D))
    return _att_call(neighs, content, attw, attb)
